# baseline (speedup 1.0000x reference)
import functools

import jax
import jax.numpy as jnp
from jax import lax
from jax.experimental import pallas as pl
from jax.experimental.pallas import tpu as pltpu
from jax.experimental.pallas import tpu_sc as plsc

N = 10000
E = 160000
D = 128
FE = 16
NPAD = 10240
EPAD = 163840
NB = 2048
EB = 2048

_INTERPRET = False


def _lrelu(t):
    return jnp.where(t > 0, t, 0.01 * t)


def _t1_body(state_ref, w1_ref, o_ref):
    o_ref[...] = jnp.dot(state_ref[...], w1_ref[...].T,
                         preferred_element_type=jnp.float32)


def _t2_body(ef_ref, g_ref, w4_ref, wap_ref, bap_ref, w1a_ref,
             x4_ref, ae_ref):
    ef = ef_ref[...]
    x4 = _lrelu(jnp.dot(ef, w4_ref[...].T, preferred_element_type=jnp.float32))
    x4_ref[...] = x4 * g_ref[...]
    a = jax.nn.relu(jnp.dot(ef, wap_ref[...].T,
                            preferred_element_type=jnp.float32) + bap_ref[...])
    ae_ref[...] = jnp.dot(a, w1a_ref[...].T, preferred_element_type=jnp.float32)


def _t3_body(msga_ref, msgb_ref, w3_ref, o_ref):
    msg = msga_ref[...] + msgb_ref[...]
    o_ref[...] = jnp.dot(msg, w3_ref[...].T, preferred_element_type=jnp.float32)


def _t4_body(h1_ref, nbra_ref, nbrb_ref, efe_ref, w2_ref, o_ref):
    nbr = nbra_ref[...] + nbrb_ref[...]
    o_ref[...] = _lrelu(h1_ref[...] +
                        jnp.dot(nbr, w2_ref[...].T,
                                preferred_element_type=jnp.float32) +
                        efe_ref[...])


def _t5_body(emb_ref, gfeat_ref, wgfc_ref, bgfc_ref, wattn_ref, battn_ref,
             wsp_ref, bsp_ref, w1g_ref, advb1_ref, vw1_ref, vb1_ref,
             vw2_ref, vb2_ref,
             gfterm_ref, c_ref, val_ref):
    emb = emb_ref[...]
    gf = jax.nn.relu(jnp.dot(gfeat_ref[...], wgfc_ref[...].T,
                             preferred_element_type=jnp.float32)
                     + bgfc_ref[...])
    wattn = wattn_ref[...]
    wa1 = wattn[:, :D]
    wa2 = wattn[:, D:]
    s_const = jnp.sum(gf * wa2) + battn_ref[0, 0]
    s = jnp.sum(emb * wa1, axis=1, keepdims=True) + s_const
    rid = lax.broadcasted_iota(jnp.int32, (NPAD, 1), 0)
    valid = rid < N
    s = jnp.where(valid, s, -1e30)
    m = jnp.max(s)
    e = jnp.where(valid, jnp.exp(s - m), 0.0)
    z = jnp.sum(e)
    pooled = jnp.sum(emb * (e / z), axis=0, keepdims=True)
    wsp = wsp_ref[...]
    wspa = wsp[:, :D]
    wspb = wsp[:, D:]
    gf_term = jnp.dot(gf, wspb.T, preferred_element_type=jnp.float32) \
        + bsp_ref[...]
    gvec = jnp.dot(pooled, wspa.T, preferred_element_type=jnp.float32) \
        + gf_term
    rg = jax.nn.relu(gvec)
    c_ref[...] = jnp.dot(rg, w1g_ref[...].T,
                         preferred_element_type=jnp.float32) + advb1_ref[...]
    hv = jax.nn.relu(jnp.dot(rg, vw1_ref[...].T,
                             preferred_element_type=jnp.float32) + vb1_ref[...])
    val_ref[...] = (jnp.sum(hv * vw2_ref[...], axis=1, keepdims=True)
                    + vb2_ref[...])
    gfterm_ref[...] = gf_term


def _t6_body(emb_ref, gfterm_ref, wspa_ref, w1i_ref, w1j_ref, ti_ref, tj_ref):
    mu = jnp.dot(emb_ref[...], wspa_ref[...].T,
                 preferred_element_type=jnp.float32) + gfterm_ref[...]
    rmu = jax.nn.relu(mu)
    ti_ref[...] = jnp.dot(rmu, w1i_ref[...].T, preferred_element_type=jnp.float32)
    tj_ref[...] = jnp.dot(rmu, w1j_ref[...].T, preferred_element_type=jnp.float32)


def _t7_body(gsum_ref, ae_ref, c_ref, w2a_ref, b2a_ref, adv_ref):
    h = jax.nn.relu(gsum_ref[...] + ae_ref[...] + c_ref[...])
    adv_ref[...] = (jnp.sum(h * w2a_ref[...], axis=1, keepdims=True)
                    + b2a_ref[0, 0])


def _t8_body(adv_ref, val_ref, q_ref):
    adv = adv_ref[...]
    rid = lax.broadcasted_iota(jnp.int32, (EPAD // 128, 128), 0)
    valid = rid < (E // 128)
    mean = jnp.sum(jnp.where(valid, adv, 0.0)) / E
    q_ref[...] = val_ref[0, 0] + adv - mean


def _full(shape):
    return pl.BlockSpec(shape, lambda *_: tuple(0 for _ in shape))


def _blocked(shape):
    n = len(shape)
    return pl.BlockSpec(shape, lambda i: (i,) + (0,) * (n - 1))


def _tc_call(body, grid, in_specs, out_specs, out_shape):
    return pl.pallas_call(
        body, grid=grid, in_specs=in_specs, out_specs=out_specs,
        out_shape=out_shape, interpret=_INTERPRET)


NC, NS, L = 2, 16, 16
NW = NC * NS
EW = EPAD // NW
NRT = NPAD // NS
GSP = 4
S1_CH = 128
S2_CH = 128
S3_CH = 128

_MESH = plsc.VectorSubcoreMesh(core_axis_name="c", subcore_axis_name="s",
                               num_cores=NC)


def _zero_vmem_rows(rows, ch):
    z = jnp.zeros((L,), jnp.float32)

    def bi(i, carry):
        for j in range(D // L):
            rows[i, pl.ds(j * L, L)] = z
        return carry

    lax.fori_loop(0, ch, bi, 0)


def _acc_zero_and_barrier(rows, acc, s, ch):
    _zero_vmem_rows(rows, ch)
    for r in range(NRT // ch):
        pltpu.sync_copy(rows, acc.at[pl.ds(s * NRT + r * ch, ch)])
    plsc.subcore_barrier()


def _acc_writeout(acc, out_hbm, c, s):
    plsc.subcore_barrier()
    pltpu.sync_copy(acc.at[pl.ds(s * NRT, NRT)],
                    out_hbm.at[c, pl.ds(s * NRT, NRT)])


def _s1_body(x4_hbm, uv_hbm, out_hbm, rows0, rows1, iuv0, iuv1, acc,
             sem0, sem1):
    ch = S1_CH
    nchk = EW // ch
    c = lax.axis_index("c")
    s = lax.axis_index("s")
    base = (s * NC + c) * EW
    rows = (rows0, rows1)
    iuv = (iuv0, iuv1)
    sems = (sem0, sem1)
    _acc_zero_and_barrier(rows0, acc, s, ch)

    def start(t, b):
        off = base + t * ch
        pltpu.sync_copy(uv_hbm.at[:, pl.ds(off, ch)], iuv[b])
        pltpu.async_copy(x4_hbm.at[pl.ds(off, ch)], rows[b], sems[b])

    start(0, 0)
    start(1, 1)

    def pair(p, carry):
        for b in range(2):
            t = 2 * p + b
            off = base + t * ch
            pltpu.make_async_copy(x4_hbm.at[pl.ds(off, ch)], rows[b],
                                  sems[b]).wait()
            pltpu.sync_copy(rows[b], acc.at[iuv[b].at[0]], add=True)
            pltpu.sync_copy(rows[b], acc.at[iuv[b].at[1]], add=True)

            @pl.when(t + 2 < nchk)
            def _():
                start(t + 2, b)
        return carry

    lax.fori_loop(0, nchk // 2, pair, 0)
    _acc_writeout(acc, out_hbm, c, s)


def _wait_split(tab_hbm, iuv_ref, rows, sem, w, ch):
    hh = ch // GSP
    for h in range(GSP):
        pltpu.make_async_copy(tab_hbm.at[iuv_ref.at[w, pl.ds(h * hh, hh)]],
                              rows.at[pl.ds(h * hh, hh)], sem).wait()


def _scale_rows(rows, gv, ch):
    def scale(k, carry2):
        gvec = gv[pl.ds(k * L, L)]
        for r in range(L):
            gb = gvec.at[jnp.full((L,), r, jnp.int32)].get(
                mode="promise_in_bounds")
            i = k * L + r
            for j in range(D // L):
                sl = pl.ds(j * L, L)
                rows[i, sl] = rows[i, sl] * gb
        return carry2

    lax.fori_loop(0, ch // L, scale, 0)


def _s2_body(emb_hbm, uv_hbm, g_hbm, out_hbm,
             ru0, ru1, iuv0, iuv1, gv0, gv1, acc, sem0, sem1):
    ch = S2_CH
    nchk = EW // ch
    c = lax.axis_index("c")
    s = lax.axis_index("s")
    base = (s * NC + c) * EW
    rows = (ru0, ru1)
    iuv = (iuv0, iuv1)
    gv = (gv0, gv1)
    sems = (sem0, sem1)
    _acc_zero_and_barrier(ru0, acc, s, ch)

    def load_idx(t, slot):
        off = base + t * ch
        pltpu.sync_copy(uv_hbm.at[:, pl.ds(off, ch)], iuv[slot])
        pltpu.sync_copy(g_hbm.at[pl.ds(off, ch)], gv[slot])

    def start_gather(t, slot, w):
        hh = ch // GSP
        for h in range(GSP):
            pltpu.async_copy(emb_hbm.at[iuv[slot].at[w, pl.ds(h * hh, hh)]],
                             rows[w].at[pl.ds(h * hh, hh)], sems[w])

    load_idx(0, 0)
    start_gather(0, 0, 0)
    start_gather(0, 0, 1)

    def pair(p, carry):
        for q in range(2):
            t = 2 * p + q
            nq = 1 - q
            _wait_split(emb_hbm, iuv[q], rows[0], sems[0], 0, ch)
            _scale_rows(rows[0], gv[q], ch)
            pltpu.sync_copy(rows[0], acc.at[iuv[q].at[1]], add=True)

            @pl.when(t + 1 < nchk)
            def _():
                load_idx(t + 1, nq)
                start_gather(t + 1, nq, 0)

            _wait_split(emb_hbm, iuv[q], rows[1], sems[1], 1, ch)
            _scale_rows(rows[1], gv[q], ch)
            pltpu.sync_copy(rows[1], acc.at[iuv[q].at[0]], add=True)

            @pl.when(t + 1 < nchk)
            def _():
                start_gather(t + 1, nq, 1)
        return carry

    lax.fori_loop(0, nchk // 2, pair, 0)
    _acc_writeout(acc, out_hbm, c, s)


def _s3_body(ti_hbm, tj_hbm, uv_hbm, out_hbm,
             ru0, ru1, rv0, rv1, iuv0, iuv1, sem0, sem1):
    ch = S3_CH
    nchk = EW // ch
    c = lax.axis_index("c")
    s = lax.axis_index("s")
    base = (s * NC + c) * EW
    ru = (ru0, ru1)
    rv = (rv0, rv1)
    iuv = (iuv0, iuv1)
    sems = (sem0, sem1)

    def start(t, b):
        off = base + t * ch
        hh = ch // GSP
        pltpu.sync_copy(uv_hbm.at[:, pl.ds(off, ch)], iuv[b])
        for h in range(GSP):
            pltpu.async_copy(ti_hbm.at[iuv[b].at[0, pl.ds(h * hh, hh)]],
                             ru[b].at[pl.ds(h * hh, hh)], sems[b])
            pltpu.async_copy(tj_hbm.at[iuv[b].at[1, pl.ds(h * hh, hh)]],
                             rv[b].at[pl.ds(h * hh, hh)], sems[b])

    start(0, 0)
    start(1, 1)

    def pair(p, carry):
        for b in range(2):
            t = 2 * p + b
            off = base + t * ch
            _wait_split(ti_hbm, iuv[b], ru[b], sems[b], 0, ch)
            _wait_split(tj_hbm, iuv[b], rv[b], sems[b], 1, ch)

            def addr(i, carry2):
                for j in range(D // L):
                    sl = pl.ds(j * L, L)
                    ru[b][i, sl] = ru[b][i, sl] + rv[b][i, sl]
                return carry2

            lax.fori_loop(0, ch, addr, 0)
            pltpu.sync_copy(ru[b], out_hbm.at[pl.ds(off, ch)])

            @pl.when(t + 2 < nchk)
            def _():
                start(t + 2, b)
        return carry

    lax.fori_loop(0, nchk // 2, pair, 0)


_f32 = jnp.float32
_i32 = jnp.int32

_s1_call = pl.kernel(
    _s1_body,
    out_type=jax.ShapeDtypeStruct((NC, NPAD, D), _f32),
    mesh=_MESH,
    scratch_types=[
        pltpu.VMEM((S1_CH, D), _f32),
        pltpu.VMEM((S1_CH, D), _f32),
        pltpu.VMEM((2, S1_CH), _i32),
        pltpu.VMEM((2, S1_CH), _i32),
        pltpu.VMEM_SHARED((NPAD, D), _f32),
        pltpu.SemaphoreType.DMA,
        pltpu.SemaphoreType.DMA,
    ],
)

_s2_call = pl.kernel(
    _s2_body,
    out_type=jax.ShapeDtypeStruct((NC, NPAD, D), _f32),
    mesh=_MESH,
    scratch_types=[
        pltpu.VMEM((S2_CH, D), _f32),
        pltpu.VMEM((S2_CH, D), _f32),
        pltpu.VMEM((2, S2_CH), _i32),
        pltpu.VMEM((2, S2_CH), _i32),
        pltpu.VMEM((S2_CH,), _f32),
        pltpu.VMEM((S2_CH,), _f32),
        pltpu.VMEM_SHARED((NPAD, D), _f32),
        pltpu.SemaphoreType.DMA,
        pltpu.SemaphoreType.DMA,
    ],
)

_s3_call = pl.kernel(
    _s3_body,
    out_type=jax.ShapeDtypeStruct((EPAD, D), _f32),
    mesh=_MESH,
    scratch_types=[
        pltpu.VMEM((S3_CH, D), _f32),
        pltpu.VMEM((S3_CH, D), _f32),
        pltpu.VMEM((S3_CH, D), _f32),
        pltpu.VMEM((S3_CH, D), _f32),
        pltpu.VMEM((2, S3_CH), _i32),
        pltpu.VMEM((2, S3_CH), _i32),
        pltpu.SemaphoreType.DMA,
        pltpu.SemaphoreType.DMA,
    ],
)


def _scatter_add_x4(x4, uv):
    return _s1_call(x4, uv)


def _round_scatter(emb, uv, g):
    return _s2_call(emb, uv, g)


def _gather_sum(ti, tj, uv):
    return _s3_call(ti, tj, uv)


def kernel(state, edge_features, edges_ij, edge_status, global_feats,
           W_theta1, W_theta2, W_theta3, W_theta4, W_gfc, b_gfc,
           W_sp, b_sp, W_ap, b_ap, W_attn, b_attn,
           adv_w1, adv_b1, adv_w2, adv_b2, val_w1, val_b1, val_w2, val_b2):
    st = jnp.pad(state[0], ((0, NPAD - N), (0, 0)))
    ef = jnp.pad(edge_features[0], ((0, EPAD - E), (0, 0)))
    u = jnp.pad(edges_ij[:, 0], (0, EPAD - E), constant_values=N)
    v = jnp.pad(edges_ij[:, 1], (0, EPAD - E), constant_values=N)
    uv = jnp.stack([u, v])
    g = jnp.pad(edge_status, (0, EPAD - E))
    g2 = g[:, None]

    w1g = adv_w1[:, :D]
    w1i = adv_w1[:, D:2 * D]
    w1j = adv_w1[:, 2 * D:3 * D]
    w1a = adv_w1[:, 3 * D:]
    wspa = W_sp[:, :D]

    f32 = jnp.float32
    ngrid = NPAD // NB
    egrid = EPAD // EB

    h1 = _tc_call(_t1_body, (ngrid,),
                  [_blocked((NB, D)), _full((D, D))],
                  _blocked((NB, D)),
                  jax.ShapeDtypeStruct((NPAD, D), f32))(st, W_theta1)

    x4, ae = _tc_call(
        _t2_body, (egrid,),
        [_blocked((EB, FE)), _blocked((EB, 1)), _full((D, FE)),
         _full((D, FE)), _full((1, D)), _full((D, D))],
        [_blocked((EB, D)), _blocked((EB, D))],
        [jax.ShapeDtypeStruct((EPAD, D), f32),
         jax.ShapeDtypeStruct((EPAD, D), f32)],
    )(ef, g2, W_theta4, W_ap, b_ap[None, :], w1a)

    msg = _scatter_add_x4(x4, uv)

    efe = _tc_call(_t3_body, (ngrid,),
                   [_blocked((NB, D)), _blocked((NB, D)), _full((D, D))],
                   _blocked((NB, D)),
                   jax.ShapeDtypeStruct((NPAD, D), f32))(msg[0], msg[1], W_theta3)

    emb = h1
    for _ in range(2):
        nbr = _round_scatter(emb, uv, g)
        emb = _tc_call(
            _t4_body, (ngrid,),
            [_blocked((NB, D))] * 4 + [_full((D, D))],
            _blocked((NB, D)),
            jax.ShapeDtypeStruct((NPAD, D), f32),
        )(h1, nbr[0], nbr[1], efe, W_theta2)

    gf_term, c, val = _tc_call(
        _t5_body, (1,),
        [_full((NPAD, D)), _full((1, 32)), _full((D, 32)), _full((1, D)),
         _full((1, 2 * D)), _full((1, 1)), _full((D, 2 * D)), _full((1, D)),
         _full((D, D)), _full((1, D)), _full((D, D)), _full((1, D)),
         _full((1, D)), _full((1, 1))],
        [_full((1, D)), _full((1, D)), _full((1, 1))],
        [jax.ShapeDtypeStruct((1, D), f32),
         jax.ShapeDtypeStruct((1, D), f32),
         jax.ShapeDtypeStruct((1, 1), f32)],
    )(emb, global_feats, W_gfc, b_gfc[None, :], W_attn, b_attn[None, :],
      W_sp, b_sp[None, :], w1g, adv_b1[None, :], val_w1, val_b1[None, :],
      val_w2, val_b2[None, :])

    ti, tj = _tc_call(
        _t6_body, (ngrid,),
        [_blocked((NB, D)), _full((1, D)), _full((D, D)), _full((D, D)),
         _full((D, D))],
        [_blocked((NB, D)), _blocked((NB, D))],
        [jax.ShapeDtypeStruct((NPAD, D), f32),
         jax.ShapeDtypeStruct((NPAD, D), f32)],
    )(emb, gf_term, wspa, w1i, w1j)

    gsum = _gather_sum(ti, tj, uv)

    adv = _tc_call(
        _t7_body, (egrid,),
        [_blocked((EB, D)), _blocked((EB, D)), _full((1, D)), _full((1, D)),
         _full((1, 1))],
        _blocked((EB, 1)),
        jax.ShapeDtypeStruct((EPAD, 1), f32),
    )(gsum, ae, c, adv_w2, adv_b2[None, :])

    q = _tc_call(
        _t8_body, (1,),
        [_full((EPAD // 128, 128)), _full((1, 1))],
        _full((EPAD // 128, 128)),
        jax.ShapeDtypeStruct((EPAD // 128, 128), f32),
    )(adv.reshape(EPAD // 128, 128), val)

    return q.reshape(EPAD)[:E][None, :]


# speedup vs baseline: 1.0178x; 1.0178x over previous
import functools

import jax
import jax.numpy as jnp
from jax import lax
from jax.experimental import pallas as pl
from jax.experimental.pallas import tpu as pltpu
from jax.experimental.pallas import tpu_sc as plsc

N = 10000
E = 160000
D = 128
FE = 16
NPAD = 10240
EPAD = 163840
NB = 2048
EB = 2048

_INTERPRET = False


def _lrelu(t):
    return jnp.where(t > 0, t, 0.01 * t)


def _t1_body(state_ref, w1_ref, o_ref):
    o_ref[...] = jnp.dot(state_ref[...], w1_ref[...].T,
                         preferred_element_type=jnp.float32)


def _t2_body(ef_ref, g_ref, w4_ref, wap_ref, bap_ref, w1a_ref,
             x4_ref, ae_ref):
    ef = ef_ref[...]
    x4 = _lrelu(jnp.dot(ef, w4_ref[...].T, preferred_element_type=jnp.float32))
    x4_ref[...] = x4 * g_ref[...]
    a = jax.nn.relu(jnp.dot(ef, wap_ref[...].T,
                            preferred_element_type=jnp.float32) + bap_ref[...])
    ae_ref[...] = jnp.dot(a, w1a_ref[...].T, preferred_element_type=jnp.float32)


def _t3_body(msga_ref, msgb_ref, w3_ref, o_ref):
    msg = msga_ref[...] + msgb_ref[...]
    o_ref[...] = jnp.dot(msg, w3_ref[...].T, preferred_element_type=jnp.float32)


def _t4_body(h1_ref, nbra_ref, nbrb_ref, efe_ref, w2_ref, o_ref):
    nbr = nbra_ref[...] + nbrb_ref[...]
    o_ref[...] = _lrelu(h1_ref[...] +
                        jnp.dot(nbr, w2_ref[...].T,
                                preferred_element_type=jnp.float32) +
                        efe_ref[...])


def _t5_body(emb_ref, gfeat_ref, wgfc_ref, bgfc_ref, wattn_ref, battn_ref,
             wsp_ref, bsp_ref, w1g_ref, advb1_ref, vw1_ref, vb1_ref,
             vw2_ref, vb2_ref,
             gfterm_ref, c_ref, val_ref):
    emb = emb_ref[...]
    gf = jax.nn.relu(jnp.dot(gfeat_ref[...], wgfc_ref[...].T,
                             preferred_element_type=jnp.float32)
                     + bgfc_ref[...])
    wattn = wattn_ref[...]
    wa1 = wattn[:, :D]
    wa2 = wattn[:, D:]
    s_const = jnp.sum(gf * wa2) + battn_ref[0, 0]
    s = jnp.sum(emb * wa1, axis=1, keepdims=True) + s_const
    rid = lax.broadcasted_iota(jnp.int32, (NPAD, 1), 0)
    valid = rid < N
    s = jnp.where(valid, s, -1e30)
    m = jnp.max(s)
    e = jnp.where(valid, jnp.exp(s - m), 0.0)
    z = jnp.sum(e)
    pooled = jnp.sum(emb * (e / z), axis=0, keepdims=True)
    wsp = wsp_ref[...]
    wspa = wsp[:, :D]
    wspb = wsp[:, D:]
    gf_term = jnp.dot(gf, wspb.T, preferred_element_type=jnp.float32) \
        + bsp_ref[...]
    gvec = jnp.dot(pooled, wspa.T, preferred_element_type=jnp.float32) \
        + gf_term
    rg = jax.nn.relu(gvec)
    c_ref[...] = jnp.dot(rg, w1g_ref[...].T,
                         preferred_element_type=jnp.float32) + advb1_ref[...]
    hv = jax.nn.relu(jnp.dot(rg, vw1_ref[...].T,
                             preferred_element_type=jnp.float32) + vb1_ref[...])
    val_ref[...] = (jnp.sum(hv * vw2_ref[...], axis=1, keepdims=True)
                    + vb2_ref[...])
    gfterm_ref[...] = gf_term


def _t6_body(emb_ref, gfterm_ref, wspa_ref, w1i_ref, w1j_ref, ti_ref, tj_ref):
    mu = jnp.dot(emb_ref[...], wspa_ref[...].T,
                 preferred_element_type=jnp.float32) + gfterm_ref[...]
    rmu = jax.nn.relu(mu)
    ti_ref[...] = jnp.dot(rmu, w1i_ref[...].T, preferred_element_type=jnp.float32)
    tj_ref[...] = jnp.dot(rmu, w1j_ref[...].T, preferred_element_type=jnp.float32)


def _t7_body(gsum_ref, ae_ref, c_ref, w2a_ref, b2a_ref, adv_ref):
    h = jax.nn.relu(gsum_ref[...] + ae_ref[...] + c_ref[...])
    adv_ref[...] = (jnp.sum(h * w2a_ref[...], axis=1, keepdims=True)
                    + b2a_ref[0, 0])


def _t8_body(adv_ref, val_ref, q_ref):
    adv = adv_ref[...]
    rid = lax.broadcasted_iota(jnp.int32, (EPAD // 128, 128), 0)
    valid = rid < (E // 128)
    mean = jnp.sum(jnp.where(valid, adv, 0.0)) / E
    q_ref[...] = val_ref[0, 0] + adv - mean


def _full(shape):
    return pl.BlockSpec(shape, lambda *_: tuple(0 for _ in shape))


def _blocked(shape):
    n = len(shape)
    return pl.BlockSpec(shape, lambda i: (i,) + (0,) * (n - 1))


def _tc_call(body, grid, in_specs, out_specs, out_shape):
    return pl.pallas_call(
        body, grid=grid, in_specs=in_specs, out_specs=out_specs,
        out_shape=out_shape, interpret=_INTERPRET)


NC, NS, L = 2, 16, 16
NW = NC * NS
EW = EPAD // NW
NRT = NPAD // NS
GSP = 4
S23_NCHK0 = 58
S23_NCHK1 = 22
CORE0_EDGES = NS * S23_NCHK0 * 128
S1_CH = 128
S2_CH = 128
S3_CH = 128

_MESH = plsc.VectorSubcoreMesh(core_axis_name="c", subcore_axis_name="s",
                               num_cores=NC)


def _zero_vmem_rows(rows, ch):
    z = jnp.zeros((L,), jnp.float32)

    def bi(i, carry):
        for j in range(D // L):
            rows[i, pl.ds(j * L, L)] = z
        return carry

    lax.fori_loop(0, ch, bi, 0)


def _acc_zero_and_barrier(rows, acc, s, ch):
    _zero_vmem_rows(rows, ch)
    for r in range(NRT // ch):
        pltpu.sync_copy(rows, acc.at[pl.ds(s * NRT + r * ch, ch)])
    plsc.subcore_barrier()


def _acc_writeout(acc, out_hbm, c, s):
    plsc.subcore_barrier()
    pltpu.sync_copy(acc.at[pl.ds(s * NRT, NRT)],
                    out_hbm.at[c, pl.ds(s * NRT, NRT)])


def _s1_body(x4_hbm, uv_hbm, out_hbm, rows0, rows1, iuv0, iuv1, acc,
             sem0, sem1):
    ch = S1_CH
    nchk = EW // ch
    c = lax.axis_index("c")
    s = lax.axis_index("s")
    base = (s * NC + c) * EW
    rows = (rows0, rows1)
    iuv = (iuv0, iuv1)
    sems = (sem0, sem1)
    _acc_zero_and_barrier(rows0, acc, s, ch)

    def start(t, b):
        off = base + t * ch
        pltpu.sync_copy(uv_hbm.at[:, pl.ds(off, ch)], iuv[b])
        pltpu.async_copy(x4_hbm.at[pl.ds(off, ch)], rows[b], sems[b])

    start(0, 0)
    start(1, 1)

    def pair(p, carry):
        for b in range(2):
            t = 2 * p + b
            off = base + t * ch
            pltpu.make_async_copy(x4_hbm.at[pl.ds(off, ch)], rows[b],
                                  sems[b]).wait()
            pltpu.sync_copy(rows[b], acc.at[iuv[b].at[0]], add=True)
            pltpu.sync_copy(rows[b], acc.at[iuv[b].at[1]], add=True)

            @pl.when(t + 2 < nchk)
            def _():
                start(t + 2, b)
        return carry

    lax.fori_loop(0, nchk // 2, pair, 0)
    _acc_writeout(acc, out_hbm, c, s)


def _wait_split(tab_hbm, iuv_ref, rows, sem, w, ch):
    hh = ch // GSP
    for h in range(GSP):
        pltpu.make_async_copy(tab_hbm.at[iuv_ref.at[w, pl.ds(h * hh, hh)]],
                              rows.at[pl.ds(h * hh, hh)], sem).wait()


def _scale_rows(rows, gv, ch):
    def scale(k, carry2):
        gvec = gv[pl.ds(k * L, L)]
        for r in range(L):
            gb = gvec.at[jnp.full((L,), r, jnp.int32)].get(
                mode="promise_in_bounds")
            i = k * L + r
            for j in range(D // L):
                sl = pl.ds(j * L, L)
                rows[i, sl] = rows[i, sl] * gb
        return carry2

    lax.fori_loop(0, ch // L, scale, 0)


def _s2_body(emb_hbm, uv_hbm, g_hbm, out_hbm,
             ru0, ru1, iuv0, iuv1, gv0, gv1, acc, sem0, sem1):
    ch = S2_CH
    c = lax.axis_index("c")
    s = lax.axis_index("s")
    nchk = S23_NCHK0 - (S23_NCHK0 - S23_NCHK1) * c
    base = c * CORE0_EDGES + s * nchk * ch
    rows = (ru0, ru1)
    iuv = (iuv0, iuv1)
    gv = (gv0, gv1)
    sems = (sem0, sem1)
    _acc_zero_and_barrier(ru0, acc, s, ch)

    def load_idx(t, slot):
        off = base + t * ch
        pltpu.sync_copy(uv_hbm.at[:, pl.ds(off, ch)], iuv[slot])
        pltpu.sync_copy(g_hbm.at[pl.ds(off, ch)], gv[slot])

    def start_gather(t, slot, w):
        hh = ch // GSP
        for h in range(GSP):
            pltpu.async_copy(emb_hbm.at[iuv[slot].at[w, pl.ds(h * hh, hh)]],
                             rows[w].at[pl.ds(h * hh, hh)], sems[w])

    load_idx(0, 0)
    start_gather(0, 0, 0)
    start_gather(0, 0, 1)

    def pair(p, carry):
        for q in range(2):
            t = 2 * p + q
            nq = 1 - q
            _wait_split(emb_hbm, iuv[q], rows[0], sems[0], 0, ch)
            _scale_rows(rows[0], gv[q], ch)
            pltpu.sync_copy(rows[0], acc.at[iuv[q].at[1]], add=True)

            @pl.when(t + 1 < nchk)
            def _():
                load_idx(t + 1, nq)
                start_gather(t + 1, nq, 0)

            _wait_split(emb_hbm, iuv[q], rows[1], sems[1], 1, ch)
            _scale_rows(rows[1], gv[q], ch)
            pltpu.sync_copy(rows[1], acc.at[iuv[q].at[0]], add=True)

            @pl.when(t + 1 < nchk)
            def _():
                start_gather(t + 1, nq, 1)
        return carry

    lax.fori_loop(0, nchk // 2, pair, 0)
    _acc_writeout(acc, out_hbm, c, s)


def _s3_body(ti_hbm, tj_hbm, uv_hbm, out_hbm,
             ru0, ru1, rv0, rv1, iuv0, iuv1, sem0, sem1):
    ch = S3_CH
    c = lax.axis_index("c")
    s = lax.axis_index("s")
    nchk = S23_NCHK0 - (S23_NCHK0 - S23_NCHK1) * c
    base = c * CORE0_EDGES + s * nchk * ch
    ru = (ru0, ru1)
    rv = (rv0, rv1)
    iuv = (iuv0, iuv1)
    sems = (sem0, sem1)

    def start(t, b):
        off = base + t * ch
        hh = ch // GSP
        pltpu.sync_copy(uv_hbm.at[:, pl.ds(off, ch)], iuv[b])
        for h in range(GSP):
            pltpu.async_copy(ti_hbm.at[iuv[b].at[0, pl.ds(h * hh, hh)]],
                             ru[b].at[pl.ds(h * hh, hh)], sems[b])
            pltpu.async_copy(tj_hbm.at[iuv[b].at[1, pl.ds(h * hh, hh)]],
                             rv[b].at[pl.ds(h * hh, hh)], sems[b])

    start(0, 0)
    start(1, 1)

    def pair(p, carry):
        for b in range(2):
            t = 2 * p + b
            off = base + t * ch
            _wait_split(ti_hbm, iuv[b], ru[b], sems[b], 0, ch)
            _wait_split(tj_hbm, iuv[b], rv[b], sems[b], 1, ch)

            def addr(i, carry2):
                for j in range(D // L):
                    sl = pl.ds(j * L, L)
                    ru[b][i, sl] = ru[b][i, sl] + rv[b][i, sl]
                return carry2

            lax.fori_loop(0, ch, addr, 0)
            pltpu.sync_copy(ru[b], out_hbm.at[pl.ds(off, ch)])

            @pl.when(t + 2 < nchk)
            def _():
                start(t + 2, b)
        return carry

    lax.fori_loop(0, nchk // 2, pair, 0)


_f32 = jnp.float32
_i32 = jnp.int32

_s1_call = pl.kernel(
    _s1_body,
    out_type=jax.ShapeDtypeStruct((NC, NPAD, D), _f32),
    mesh=_MESH,
    scratch_types=[
        pltpu.VMEM((S1_CH, D), _f32),
        pltpu.VMEM((S1_CH, D), _f32),
        pltpu.VMEM((2, S1_CH), _i32),
        pltpu.VMEM((2, S1_CH), _i32),
        pltpu.VMEM_SHARED((NPAD, D), _f32),
        pltpu.SemaphoreType.DMA,
        pltpu.SemaphoreType.DMA,
    ],
)

_s2_call = pl.kernel(
    _s2_body,
    out_type=jax.ShapeDtypeStruct((NC, NPAD, D), _f32),
    mesh=_MESH,
    scratch_types=[
        pltpu.VMEM((S2_CH, D), _f32),
        pltpu.VMEM((S2_CH, D), _f32),
        pltpu.VMEM((2, S2_CH), _i32),
        pltpu.VMEM((2, S2_CH), _i32),
        pltpu.VMEM((S2_CH,), _f32),
        pltpu.VMEM((S2_CH,), _f32),
        pltpu.VMEM_SHARED((NPAD, D), _f32),
        pltpu.SemaphoreType.DMA,
        pltpu.SemaphoreType.DMA,
    ],
)

_s3_call = pl.kernel(
    _s3_body,
    out_type=jax.ShapeDtypeStruct((EPAD, D), _f32),
    mesh=_MESH,
    scratch_types=[
        pltpu.VMEM((S3_CH, D), _f32),
        pltpu.VMEM((S3_CH, D), _f32),
        pltpu.VMEM((S3_CH, D), _f32),
        pltpu.VMEM((S3_CH, D), _f32),
        pltpu.VMEM((2, S3_CH), _i32),
        pltpu.VMEM((2, S3_CH), _i32),
        pltpu.SemaphoreType.DMA,
        pltpu.SemaphoreType.DMA,
    ],
)


def _scatter_add_x4(x4, uv):
    return _s1_call(x4, uv)


def _round_scatter(emb, uv, g):
    return _s2_call(emb, uv, g)


def _gather_sum(ti, tj, uv):
    return _s3_call(ti, tj, uv)


def kernel(state, edge_features, edges_ij, edge_status, global_feats,
           W_theta1, W_theta2, W_theta3, W_theta4, W_gfc, b_gfc,
           W_sp, b_sp, W_ap, b_ap, W_attn, b_attn,
           adv_w1, adv_b1, adv_w2, adv_b2, val_w1, val_b1, val_w2, val_b2):
    st = jnp.pad(state[0], ((0, NPAD - N), (0, 0)))
    ef = jnp.pad(edge_features[0], ((0, EPAD - E), (0, 0)))
    u = jnp.pad(edges_ij[:, 0], (0, EPAD - E), constant_values=N)
    v = jnp.pad(edges_ij[:, 1], (0, EPAD - E), constant_values=N)
    uv = jnp.stack([u, v])
    g = jnp.pad(edge_status, (0, EPAD - E))
    g2 = g[:, None]

    w1g = adv_w1[:, :D]
    w1i = adv_w1[:, D:2 * D]
    w1j = adv_w1[:, 2 * D:3 * D]
    w1a = adv_w1[:, 3 * D:]
    wspa = W_sp[:, :D]

    f32 = jnp.float32
    ngrid = NPAD // NB
    egrid = EPAD // EB

    h1 = _tc_call(_t1_body, (ngrid,),
                  [_blocked((NB, D)), _full((D, D))],
                  _blocked((NB, D)),
                  jax.ShapeDtypeStruct((NPAD, D), f32))(st, W_theta1)

    x4, ae = _tc_call(
        _t2_body, (egrid,),
        [_blocked((EB, FE)), _blocked((EB, 1)), _full((D, FE)),
         _full((D, FE)), _full((1, D)), _full((D, D))],
        [_blocked((EB, D)), _blocked((EB, D))],
        [jax.ShapeDtypeStruct((EPAD, D), f32),
         jax.ShapeDtypeStruct((EPAD, D), f32)],
    )(ef, g2, W_theta4, W_ap, b_ap[None, :], w1a)

    msg = _scatter_add_x4(x4, uv)

    efe = _tc_call(_t3_body, (ngrid,),
                   [_blocked((NB, D)), _blocked((NB, D)), _full((D, D))],
                   _blocked((NB, D)),
                   jax.ShapeDtypeStruct((NPAD, D), f32))(msg[0], msg[1], W_theta3)

    emb = h1
    for _ in range(2):
        nbr = _round_scatter(emb, uv, g)
        emb = _tc_call(
            _t4_body, (ngrid,),
            [_blocked((NB, D))] * 4 + [_full((D, D))],
            _blocked((NB, D)),
            jax.ShapeDtypeStruct((NPAD, D), f32),
        )(h1, nbr[0], nbr[1], efe, W_theta2)

    gf_term, c, val = _tc_call(
        _t5_body, (1,),
        [_full((NPAD, D)), _full((1, 32)), _full((D, 32)), _full((1, D)),
         _full((1, 2 * D)), _full((1, 1)), _full((D, 2 * D)), _full((1, D)),
         _full((D, D)), _full((1, D)), _full((D, D)), _full((1, D)),
         _full((1, D)), _full((1, 1))],
        [_full((1, D)), _full((1, D)), _full((1, 1))],
        [jax.ShapeDtypeStruct((1, D), f32),
         jax.ShapeDtypeStruct((1, D), f32),
         jax.ShapeDtypeStruct((1, 1), f32)],
    )(emb, global_feats, W_gfc, b_gfc[None, :], W_attn, b_attn[None, :],
      W_sp, b_sp[None, :], w1g, adv_b1[None, :], val_w1, val_b1[None, :],
      val_w2, val_b2[None, :])

    ti, tj = _tc_call(
        _t6_body, (ngrid,),
        [_blocked((NB, D)), _full((1, D)), _full((D, D)), _full((D, D)),
         _full((D, D))],
        [_blocked((NB, D)), _blocked((NB, D))],
        [jax.ShapeDtypeStruct((NPAD, D), f32),
         jax.ShapeDtypeStruct((NPAD, D), f32)],
    )(emb, gf_term, wspa, w1i, w1j)

    gsum = _gather_sum(ti, tj, uv)

    adv = _tc_call(
        _t7_body, (egrid,),
        [_blocked((EB, D)), _blocked((EB, D)), _full((1, D)), _full((1, D)),
         _full((1, 1))],
        _blocked((EB, 1)),
        jax.ShapeDtypeStruct((EPAD, 1), f32),
    )(gsum, ae, c, adv_w2, adv_b2[None, :])

    q = _tc_call(
        _t8_body, (1,),
        [_full((EPAD // 128, 128)), _full((1, 1))],
        _full((EPAD // 128, 128)),
        jax.ShapeDtypeStruct((EPAD // 128, 128), f32),
    )(adv.reshape(EPAD // 128, 128), val)

    return q.reshape(EPAD)[:E][None, :]


# speedup vs baseline: 1.0305x; 1.0124x over previous
import functools

import jax
import jax.numpy as jnp
from jax import lax
from jax.experimental import pallas as pl
from jax.experimental.pallas import tpu as pltpu
from jax.experimental.pallas import tpu_sc as plsc

N = 10000
E = 160000
D = 128
FE = 16
NPAD = 10240
EPAD = 163840
NB = 2048
EB = 2048

_INTERPRET = False


def _lrelu(t):
    return jnp.where(t > 0, t, 0.01 * t)


def _t1_body(state_ref, w1_ref, o_ref):
    o_ref[...] = jnp.dot(state_ref[...], w1_ref[...].T,
                         preferred_element_type=jnp.float32, precision=lax.Precision.HIGHEST)


def _t2_body(ef_ref, g_ref, w4_ref, wap_ref, bap_ref, w1a_ref,
             x4_ref, ae_ref):
    ef = ef_ref[...]
    x4 = _lrelu(jnp.dot(ef, w4_ref[...].T, preferred_element_type=jnp.float32, precision=lax.Precision.HIGHEST))
    x4_ref[...] = x4 * g_ref[...]
    a = jax.nn.relu(jnp.dot(ef, wap_ref[...].T,
                            preferred_element_type=jnp.float32, precision=lax.Precision.HIGHEST) + bap_ref[...])
    ae_ref[...] = jnp.dot(a, w1a_ref[...].T, preferred_element_type=jnp.float32, precision=lax.Precision.HIGHEST)


def _t3_body(msga_ref, msgb_ref, w3_ref, o_ref):
    msg = msga_ref[...] + msgb_ref[...]
    o_ref[...] = jnp.dot(msg, w3_ref[...].T, preferred_element_type=jnp.float32, precision=lax.Precision.HIGHEST)


def _t4_body(h1_ref, nbra_ref, nbrb_ref, efe_ref, w2_ref, o_ref):
    nbr = nbra_ref[...] + nbrb_ref[...]
    o_ref[...] = _lrelu(h1_ref[...] +
                        jnp.dot(nbr, w2_ref[...].T,
                                preferred_element_type=jnp.float32, precision=lax.Precision.HIGHEST) +
                        efe_ref[...])


def _t5_body(emb_ref, gfeat_ref, wgfc_ref, bgfc_ref, wattn_ref, battn_ref,
             wsp_ref, bsp_ref, w1g_ref, advb1_ref, vw1_ref, vb1_ref,
             vw2_ref, vb2_ref,
             gfterm_ref, c_ref, val_ref):
    emb = emb_ref[...]
    gf = jax.nn.relu(jnp.dot(gfeat_ref[...], wgfc_ref[...].T,
                             preferred_element_type=jnp.float32, precision=lax.Precision.HIGHEST)
                     + bgfc_ref[...])
    wattn = wattn_ref[...]
    wa1 = wattn[:, :D]
    wa2 = wattn[:, D:]
    s_const = jnp.sum(gf * wa2) + battn_ref[0, 0]
    s = jnp.sum(emb * wa1, axis=1, keepdims=True) + s_const
    rid = lax.broadcasted_iota(jnp.int32, (NPAD, 1), 0)
    valid = rid < N
    s = jnp.where(valid, s, -1e30)
    m = jnp.max(s)
    e = jnp.where(valid, jnp.exp(s - m), 0.0)
    z = jnp.sum(e)
    pooled = jnp.sum(emb * (e / z), axis=0, keepdims=True)
    wsp = wsp_ref[...]
    wspa = wsp[:, :D]
    wspb = wsp[:, D:]
    gf_term = jnp.dot(gf, wspb.T, preferred_element_type=jnp.float32, precision=lax.Precision.HIGHEST) \
        + bsp_ref[...]
    gvec = jnp.dot(pooled, wspa.T, preferred_element_type=jnp.float32, precision=lax.Precision.HIGHEST) \
        + gf_term
    rg = jax.nn.relu(gvec)
    c_ref[...] = jnp.dot(rg, w1g_ref[...].T,
                         preferred_element_type=jnp.float32, precision=lax.Precision.HIGHEST) + advb1_ref[...]
    hv = jax.nn.relu(jnp.dot(rg, vw1_ref[...].T,
                             preferred_element_type=jnp.float32, precision=lax.Precision.HIGHEST) + vb1_ref[...])
    val_ref[...] = (jnp.sum(hv * vw2_ref[...], axis=1, keepdims=True)
                    + vb2_ref[...])
    gfterm_ref[...] = gf_term


def _t6_body(emb_ref, gfterm_ref, wspa_ref, w1i_ref, w1j_ref, ti_ref, tj_ref):
    mu = jnp.dot(emb_ref[...], wspa_ref[...].T,
                 preferred_element_type=jnp.float32, precision=lax.Precision.HIGHEST) + gfterm_ref[...]
    rmu = jax.nn.relu(mu)
    ti_ref[...] = jnp.dot(rmu, w1i_ref[...].T, preferred_element_type=jnp.float32, precision=lax.Precision.HIGHEST)
    tj_ref[...] = jnp.dot(rmu, w1j_ref[...].T, preferred_element_type=jnp.float32, precision=lax.Precision.HIGHEST)


def _t7_body(gsum_ref, ae_ref, c_ref, w2a_ref, b2a_ref, adv_ref):
    h = jax.nn.relu(gsum_ref[...] + ae_ref[...] + c_ref[...])
    adv_ref[...] = (jnp.sum(h * w2a_ref[...], axis=1, keepdims=True)
                    + b2a_ref[0, 0])


def _t8_body(adv_ref, val_ref, q_ref):
    adv = adv_ref[...]
    rid = lax.broadcasted_iota(jnp.int32, (EPAD // 128, 128), 0)
    valid = rid < (E // 128)
    mean = jnp.sum(jnp.where(valid, adv, 0.0)) / E
    q_ref[...] = val_ref[0, 0] + adv - mean


def _full(shape):
    return pl.BlockSpec(shape, lambda *_: tuple(0 for _ in shape))


def _blocked(shape):
    n = len(shape)
    return pl.BlockSpec(shape, lambda i: (i,) + (0,) * (n - 1))


def _tc_call(body, grid, in_specs, out_specs, out_shape):
    return pl.pallas_call(
        body, grid=grid, in_specs=in_specs, out_specs=out_specs,
        out_shape=out_shape, interpret=_INTERPRET)


NC, NS, L = 2, 16, 16
NW = NC * NS
EW = EPAD // NW
NRT = NPAD // NS
GSP = 4
S23_NCHK0 = 58
S23_NCHK1 = 22
CORE0_EDGES = NS * S23_NCHK0 * 128
S1_CH = 128
S2_CH = 128
S3_CH = 128

_MESH = plsc.VectorSubcoreMesh(core_axis_name="c", subcore_axis_name="s",
                               num_cores=NC)


def _zero_vmem_rows(rows, ch):
    z = jnp.zeros((L,), jnp.float32)

    def bi(i, carry):
        for j in range(D // L):
            rows[i, pl.ds(j * L, L)] = z
        return carry

    lax.fori_loop(0, ch, bi, 0)


def _acc_zero_and_barrier(rows, acc, s, ch):
    _zero_vmem_rows(rows, ch)
    for r in range(NRT // ch):
        pltpu.sync_copy(rows, acc.at[pl.ds(s * NRT + r * ch, ch)])
    plsc.subcore_barrier()


def _acc_writeout(acc, out_hbm, c, s):
    plsc.subcore_barrier()
    pltpu.sync_copy(acc.at[pl.ds(s * NRT, NRT)],
                    out_hbm.at[c, pl.ds(s * NRT, NRT)])


def _s1_body(x4_hbm, uv_hbm, out_hbm, rows0, rows1, iuv0, iuv1, acc,
             sem0, sem1):
    ch = S1_CH
    nchk = EW // ch
    c = lax.axis_index("c")
    s = lax.axis_index("s")
    base = (s * NC + c) * EW
    rows = (rows0, rows1)
    iuv = (iuv0, iuv1)
    sems = (sem0, sem1)
    _acc_zero_and_barrier(rows0, acc, s, ch)

    def start(t, b):
        off = base + t * ch
        pltpu.sync_copy(uv_hbm.at[:, pl.ds(off, ch)], iuv[b])
        pltpu.async_copy(x4_hbm.at[pl.ds(off, ch)], rows[b], sems[b])

    start(0, 0)
    start(1, 1)

    def pair(p, carry):
        for b in range(2):
            t = 2 * p + b
            off = base + t * ch
            pltpu.make_async_copy(x4_hbm.at[pl.ds(off, ch)], rows[b],
                                  sems[b]).wait()
            pltpu.sync_copy(rows[b], acc.at[iuv[b].at[0]], add=True)
            pltpu.sync_copy(rows[b], acc.at[iuv[b].at[1]], add=True)

            @pl.when(t + 2 < nchk)
            def _():
                start(t + 2, b)
        return carry

    lax.fori_loop(0, nchk // 2, pair, 0)
    _acc_writeout(acc, out_hbm, c, s)


def _wait_split(tab_hbm, iuv_ref, rows, sem, w, ch):
    hh = ch // GSP
    for h in range(GSP):
        pltpu.make_async_copy(tab_hbm.at[iuv_ref.at[w, pl.ds(h * hh, hh)]],
                              rows.at[pl.ds(h * hh, hh)], sem).wait()


def _scale_rows(rows, gv, ch):
    def scale(k, carry2):
        gvec = gv[pl.ds(k * L, L)]
        for r in range(L):
            gb = gvec.at[jnp.full((L,), r, jnp.int32)].get(
                mode="promise_in_bounds")
            i = k * L + r
            for j in range(D // L):
                sl = pl.ds(j * L, L)
                rows[i, sl] = rows[i, sl] * gb
        return carry2

    lax.fori_loop(0, ch // L, scale, 0)


def _s2_body(emb_hbm, uv_hbm, g_hbm, out_hbm,
             ru0, ru1, iuv0, iuv1, gv0, gv1, acc, sem0, sem1):
    ch = S2_CH
    c = lax.axis_index("c")
    s = lax.axis_index("s")
    nchk = S23_NCHK0 - (S23_NCHK0 - S23_NCHK1) * c
    base = c * CORE0_EDGES + s * nchk * ch
    rows = (ru0, ru1)
    iuv = (iuv0, iuv1)
    gv = (gv0, gv1)
    sems = (sem0, sem1)
    _acc_zero_and_barrier(ru0, acc, s, ch)

    def load_idx(t, slot):
        off = base + t * ch
        pltpu.sync_copy(uv_hbm.at[:, pl.ds(off, ch)], iuv[slot])
        pltpu.sync_copy(g_hbm.at[pl.ds(off, ch)], gv[slot])

    def start_gather(t, slot, w):
        hh = ch // GSP
        for h in range(GSP):
            pltpu.async_copy(emb_hbm.at[iuv[slot].at[w, pl.ds(h * hh, hh)]],
                             rows[w].at[pl.ds(h * hh, hh)], sems[w])

    load_idx(0, 0)
    start_gather(0, 0, 0)
    start_gather(0, 0, 1)

    def pair(p, carry):
        for q in range(2):
            t = 2 * p + q
            nq = 1 - q
            _wait_split(emb_hbm, iuv[q], rows[0], sems[0], 0, ch)
            _scale_rows(rows[0], gv[q], ch)
            pltpu.sync_copy(rows[0], acc.at[iuv[q].at[1]], add=True)

            @pl.when(t + 1 < nchk)
            def _():
                load_idx(t + 1, nq)
                start_gather(t + 1, nq, 0)

            _wait_split(emb_hbm, iuv[q], rows[1], sems[1], 1, ch)
            _scale_rows(rows[1], gv[q], ch)
            pltpu.sync_copy(rows[1], acc.at[iuv[q].at[0]], add=True)

            @pl.when(t + 1 < nchk)
            def _():
                start_gather(t + 1, nq, 1)
        return carry

    lax.fori_loop(0, nchk // 2, pair, 0)
    _acc_writeout(acc, out_hbm, c, s)


def _s3_body(ti_hbm, tj_hbm, uv_hbm, out_hbm,
             ru0, ru1, rv0, rv1, iuv0, iuv1, sem0, sem1):
    ch = S3_CH
    c = lax.axis_index("c")
    s = lax.axis_index("s")
    nchk = S23_NCHK0 - (S23_NCHK0 - S23_NCHK1) * c
    base = c * CORE0_EDGES + s * nchk * ch
    ru = (ru0, ru1)
    rv = (rv0, rv1)
    iuv = (iuv0, iuv1)
    sems = (sem0, sem1)

    def start(t, b):
        off = base + t * ch
        hh = ch // GSP
        pltpu.sync_copy(uv_hbm.at[:, pl.ds(off, ch)], iuv[b])
        for h in range(GSP):
            pltpu.async_copy(ti_hbm.at[iuv[b].at[0, pl.ds(h * hh, hh)]],
                             ru[b].at[pl.ds(h * hh, hh)], sems[b])
            pltpu.async_copy(tj_hbm.at[iuv[b].at[1, pl.ds(h * hh, hh)]],
                             rv[b].at[pl.ds(h * hh, hh)], sems[b])

    start(0, 0)
    start(1, 1)

    def pair(p, carry):
        for b in range(2):
            t = 2 * p + b
            off = base + t * ch
            _wait_split(ti_hbm, iuv[b], ru[b], sems[b], 0, ch)
            _wait_split(tj_hbm, iuv[b], rv[b], sems[b], 1, ch)

            def addr(i, carry2):
                for j in range(D // L):
                    sl = pl.ds(j * L, L)
                    ru[b][i, sl] = ru[b][i, sl] + rv[b][i, sl]
                return carry2

            lax.fori_loop(0, ch, addr, 0)
            pltpu.sync_copy(ru[b], out_hbm.at[pl.ds(off, ch)])

            @pl.when(t + 2 < nchk)
            def _():
                start(t + 2, b)
        return carry

    lax.fori_loop(0, nchk // 2, pair, 0)


_f32 = jnp.float32
_i32 = jnp.int32

_s1_call = pl.kernel(
    _s1_body,
    out_type=jax.ShapeDtypeStruct((NC, NPAD, D), _f32),
    mesh=_MESH,
    scratch_types=[
        pltpu.VMEM((S1_CH, D), _f32),
        pltpu.VMEM((S1_CH, D), _f32),
        pltpu.VMEM((2, S1_CH), _i32),
        pltpu.VMEM((2, S1_CH), _i32),
        pltpu.VMEM_SHARED((NPAD, D), _f32),
        pltpu.SemaphoreType.DMA,
        pltpu.SemaphoreType.DMA,
    ],
)

_s2_call = pl.kernel(
    _s2_body,
    out_type=jax.ShapeDtypeStruct((NC, NPAD, D), _f32),
    mesh=_MESH,
    scratch_types=[
        pltpu.VMEM((S2_CH, D), _f32),
        pltpu.VMEM((S2_CH, D), _f32),
        pltpu.VMEM((2, S2_CH), _i32),
        pltpu.VMEM((2, S2_CH), _i32),
        pltpu.VMEM((S2_CH,), _f32),
        pltpu.VMEM((S2_CH,), _f32),
        pltpu.VMEM_SHARED((NPAD, D), _f32),
        pltpu.SemaphoreType.DMA,
        pltpu.SemaphoreType.DMA,
    ],
)

_s3_call = pl.kernel(
    _s3_body,
    out_type=jax.ShapeDtypeStruct((EPAD, D), _f32),
    mesh=_MESH,
    scratch_types=[
        pltpu.VMEM((S3_CH, D), _f32),
        pltpu.VMEM((S3_CH, D), _f32),
        pltpu.VMEM((S3_CH, D), _f32),
        pltpu.VMEM((S3_CH, D), _f32),
        pltpu.VMEM((2, S3_CH), _i32),
        pltpu.VMEM((2, S3_CH), _i32),
        pltpu.SemaphoreType.DMA,
        pltpu.SemaphoreType.DMA,
    ],
)


def _scatter_add_x4(x4, uv):
    return _s1_call(x4, uv)


def _round_scatter(emb, uv, g):
    return _s2_call(emb, uv, g)


def _gather_sum(ti, tj, uv):
    return _s3_call(ti, tj, uv)


def kernel(state, edge_features, edges_ij, edge_status, global_feats,
           W_theta1, W_theta2, W_theta3, W_theta4, W_gfc, b_gfc,
           W_sp, b_sp, W_ap, b_ap, W_attn, b_attn,
           adv_w1, adv_b1, adv_w2, adv_b2, val_w1, val_b1, val_w2, val_b2):
    st = jnp.pad(state[0], ((0, NPAD - N), (0, 0)))
    ef = jnp.pad(edge_features[0], ((0, EPAD - E), (0, 0)))
    u = jnp.pad(edges_ij[:, 0], (0, EPAD - E), constant_values=N)
    v = jnp.pad(edges_ij[:, 1], (0, EPAD - E), constant_values=N)
    uv = jnp.stack([u, v])
    g = jnp.pad(edge_status, (0, EPAD - E))
    g2 = g[:, None]

    w1g = adv_w1[:, :D]
    w1i = adv_w1[:, D:2 * D]
    w1j = adv_w1[:, 2 * D:3 * D]
    w1a = adv_w1[:, 3 * D:]
    wspa = W_sp[:, :D]

    f32 = jnp.float32
    ngrid = NPAD // NB
    egrid = EPAD // EB

    h1 = _tc_call(_t1_body, (ngrid,),
                  [_blocked((NB, D)), _full((D, D))],
                  _blocked((NB, D)),
                  jax.ShapeDtypeStruct((NPAD, D), f32))(st, W_theta1)

    x4, ae = _tc_call(
        _t2_body, (egrid,),
        [_blocked((EB, FE)), _blocked((EB, 1)), _full((D, FE)),
         _full((D, FE)), _full((1, D)), _full((D, D))],
        [_blocked((EB, D)), _blocked((EB, D))],
        [jax.ShapeDtypeStruct((EPAD, D), f32),
         jax.ShapeDtypeStruct((EPAD, D), f32)],
    )(ef, g2, W_theta4, W_ap, b_ap[None, :], w1a)

    msg = _scatter_add_x4(x4, uv)

    efe = _tc_call(_t3_body, (ngrid,),
                   [_blocked((NB, D)), _blocked((NB, D)), _full((D, D))],
                   _blocked((NB, D)),
                   jax.ShapeDtypeStruct((NPAD, D), f32))(msg[0], msg[1], W_theta3)

    emb = h1
    for _ in range(2):
        nbr = _round_scatter(emb, uv, g)
        emb = _tc_call(
            _t4_body, (ngrid,),
            [_blocked((NB, D))] * 4 + [_full((D, D))],
            _blocked((NB, D)),
            jax.ShapeDtypeStruct((NPAD, D), f32),
        )(h1, nbr[0], nbr[1], efe, W_theta2)

    gf_term, c, val = _tc_call(
        _t5_body, (1,),
        [_full((NPAD, D)), _full((1, 32)), _full((D, 32)), _full((1, D)),
         _full((1, 2 * D)), _full((1, 1)), _full((D, 2 * D)), _full((1, D)),
         _full((D, D)), _full((1, D)), _full((D, D)), _full((1, D)),
         _full((1, D)), _full((1, 1))],
        [_full((1, D)), _full((1, D)), _full((1, 1))],
        [jax.ShapeDtypeStruct((1, D), f32),
         jax.ShapeDtypeStruct((1, D), f32),
         jax.ShapeDtypeStruct((1, 1), f32)],
    )(emb, global_feats, W_gfc, b_gfc[None, :], W_attn, b_attn[None, :],
      W_sp, b_sp[None, :], w1g, adv_b1[None, :], val_w1, val_b1[None, :],
      val_w2, val_b2[None, :])

    ti, tj = _tc_call(
        _t6_body, (ngrid,),
        [_blocked((NB, D)), _full((1, D)), _full((D, D)), _full((D, D)),
         _full((D, D))],
        [_blocked((NB, D)), _blocked((NB, D))],
        [jax.ShapeDtypeStruct((NPAD, D), f32),
         jax.ShapeDtypeStruct((NPAD, D), f32)],
    )(emb, gf_term, wspa, w1i, w1j)

    gsum = _gather_sum(ti, tj, uv)

    adv = _tc_call(
        _t7_body, (egrid,),
        [_blocked((EB, D)), _blocked((EB, D)), _full((1, D)), _full((1, D)),
         _full((1, 1))],
        _blocked((EB, 1)),
        jax.ShapeDtypeStruct((EPAD, 1), f32),
    )(gsum, ae, c, adv_w2, adv_b2[None, :])

    q = _tc_call(
        _t8_body, (1,),
        [_full((EPAD // 128, 128)), _full((1, 1))],
        _full((EPAD // 128, 128)),
        jax.ShapeDtypeStruct((EPAD // 128, 128), f32),
    )(adv.reshape(EPAD // 128, 128), val)

    return q.reshape(EPAD)[:E][None, :]
